# baseline (speedup 1.0000x reference)
import functools
import math

import jax
import jax.numpy as jnp
from jax import lax
from jax.experimental import pallas as pl
from jax.experimental.pallas import tpu as pltpu
from jax.experimental.pallas import tpu_sc as plsc

VOCAB = 1_000_000
D = 64
SCALE = math.sqrt(D)

_info = plsc.get_sparse_core_info()
NC = _info.num_cores
NS = _info.num_subcores
NW = NC * NS
L = _info.num_lanes

CHUNK = 128


def _build(B):
    per_w = B // NW
    nch = per_w // CHUNK

    mesh = plsc.VectorSubcoreMesh(core_axis_name="c", subcore_axis_name="s")

    @functools.partial(
        pl.kernel,
        mesh=mesh,
        compiler_params=pltpu.CompilerParams(use_tc_tiling_on_sc=False),
        out_type=jax.ShapeDtypeStruct((B, D), jnp.float32),
        scratch_types=[
            pltpu.VMEM((nch, CHUNK), jnp.int32),
            pltpu.VMEM((CHUNK, D), jnp.float32),
            pltpu.SemaphoreType.DMA,
        ],
    )
    def emb(tok_hbm, table_hbm, out_hbm, idx_v, rows_v, sem):
        wid = lax.axis_index("s") * NC + lax.axis_index("c")
        pltpu.sync_copy(tok_hbm.at[pl.ds(wid * nch, nch)], idx_v)
        row0 = wid * per_w

        def chunk_body(j, _):
            pltpu.async_copy(table_hbm.at[idx_v.at[j]], rows_v, sem).wait()

            def scale_body(r, _):
                for v in range(D // L):
                    sl = pl.ds(v * L, L)
                    rows_v[r, sl] = rows_v[r, sl] * SCALE
                return ()

            lax.fori_loop(0, CHUNK, scale_body, ())
            pltpu.sync_copy(rows_v, out_hbm.at[pl.ds(row0 + j * CHUNK, CHUNK)])
            return ()

        lax.fori_loop(0, nch, chunk_body, ())

    return emb


def kernel(tokens, table):
    S, T = tokens.shape
    B = S * T
    tok2d = tokens.reshape(B // CHUNK, CHUNK).astype(jnp.int32)
    out = _build(B)(tok2d, table)
    return out.reshape(S, T, D)
